# Initial kernel scaffold; baseline (speedup 1.0000x reference)
import jax
import jax.numpy as jnp
from jax.experimental import pallas as pl


def kernel(h, nodeset, nb_nodes, nb_weights, Q_weight, Q_bias, W_weight, W_bias):
    raise NotImplementedError("write your pallas kernel here")


# speedup vs baseline: 1.1641x; 1.1641x over previous
import functools

import jax
import jax.numpy as jnp
from jax import lax
from jax.experimental import pallas as pl
from jax.experimental.pallas import tpu as pltpu
from jax.experimental.pallas import tpu_sc as plsc

F = 128
NC, NS = 2, 16
NW = NC * NS
BPAD = 10240
T = 32
BPW = BPAD // NW
CH = 16
NCHUNK = BPW // CH


def _q_body(h_ref, qt_ref, qb_ref, out_ref):
    y = jnp.dot(h_ref[...], qt_ref[...], preferred_element_type=jnp.float32)
    y = y + qb_ref[...]
    out_ref[...] = jnp.where(y >= 0, y, 0.01 * y)


def _q_transform(h, q_t, q_bias):
    n = h.shape[0]
    blk = 1000
    grid = (n // blk,)
    return pl.pallas_call(
        _q_body,
        grid=grid,
        in_specs=[
            pl.BlockSpec((blk, F), lambda i: (i, 0)),
            pl.BlockSpec((F, F), lambda i: (0, 0)),
            pl.BlockSpec((1, F), lambda i: (0, 0)),
        ],
        out_specs=pl.BlockSpec((blk, F), lambda i: (i, 0)),
        out_shape=jax.ShapeDtypeStruct((n, F), jnp.float32),
    )(h, q_t, q_bias)


def _sc_body(hq_hbm, h_hbm, nbidx_hbm, w_hbm, ns_hbm,
             agg_hbm, self_hbm,
             idx_v, w_v, ns_v, rows_v, out_v, self_v, sem, sem2):
    wid = lax.axis_index("s") * NC + lax.axis_index("c")
    ebase = wid * (BPW * T)
    rbase = wid * BPW
    pltpu.sync_copy(nbidx_hbm.at[pl.ds(ebase, BPW * T)], idx_v)
    pltpu.sync_copy(w_hbm.at[pl.ds(ebase, BPW * T)], w_v)
    pltpu.sync_copy(ns_hbm.at[pl.ds(rbase, BPW)], ns_v)

    for g in range(NCHUNK):
        cbase = g * (CH * T)
        cps = [
            pltpu.async_copy(
                hq_hbm.at[idx_v.at[pl.ds(cbase + j * 128, 128)]],
                rows_v.at[pl.ds(j * 128, 128)],
                sem,
            )
            for j in range((CH * T) // 128)
        ]
        scp = pltpu.async_copy(
            h_hbm.at[ns_v.at[pl.ds(g * CH, CH)]], self_v, sem2)
        for c in cps:
            c.wait()

        def row_body(i, _):
            eb = cbase + i * T

            def t_body(t, acc):
                half = t // 16
                wvec = w_v[pl.ds(eb + half * 16, 16)]
                wb = lax.gather(
                    wvec,
                    jnp.full((16, 1), t % 16, jnp.int32),
                    lax.GatherDimensionNumbers(
                        offset_dims=(), collapsed_slice_dims=(0,),
                        start_index_map=(0,)),
                    slice_sizes=(1,),
                    mode=lax.GatherScatterMode.PROMISE_IN_BOUNDS)
                r = i * T + t
                return tuple(
                    acc[k] + wb * rows_v[r, pl.ds(k * 16, 16)]
                    for k in range(8))

            acc = lax.fori_loop(
                0, T, t_body,
                tuple(jnp.zeros((16,), jnp.float32) for _ in range(8)))
            for k in range(8):
                out_v[i, pl.ds(k * 16, 16)] = acc[k]
            return 0

        lax.fori_loop(0, CH, row_body, 0)
        scp.wait()
        pltpu.sync_copy(out_v, agg_hbm.at[pl.ds(rbase + g * CH, CH)])
        pltpu.sync_copy(self_v, self_hbm.at[pl.ds(rbase + g * CH, CH)])


def _sc_gather_agg(hq, h, nbidx_flat, w_flat, ns_pad):
    mesh = plsc.VectorSubcoreMesh(core_axis_name="c", subcore_axis_name="s")
    kern = functools.partial(
        pl.kernel,
        mesh=mesh,
        out_type=(
            jax.ShapeDtypeStruct((BPAD, F), jnp.float32),
            jax.ShapeDtypeStruct((BPAD, F), jnp.float32),
        ),
        scratch_types=[
            pltpu.VMEM((BPW * T,), jnp.int32),
            pltpu.VMEM((BPW * T,), jnp.float32),
            pltpu.VMEM((BPW,), jnp.int32),
            pltpu.VMEM((CH * T, F), jnp.float32),
            pltpu.VMEM((CH, F), jnp.float32),
            pltpu.VMEM((CH, F), jnp.float32),
            pltpu.SemaphoreType.DMA,
            pltpu.SemaphoreType.DMA,
        ],
    )(_sc_body)
    return kern(hq, h, nbidx_flat, w_flat, ns_pad)


def _w_body(self_ref, agg_ref, w_ref, w1t_ref, w2t_ref, wb_ref, out_ref):
    wsum = jnp.sum(w_ref[...], axis=1, keepdims=True)
    wsum = jnp.where(wsum == 0, 1.0, wsum)
    y = jnp.dot(self_ref[...], w1t_ref[...], preferred_element_type=jnp.float32)
    y2 = jnp.dot(agg_ref[...], w2t_ref[...], preferred_element_type=jnp.float32)
    y = y + y2 / wsum + wb_ref[...]
    y = jnp.where(y >= 0, y, 0.01 * y)
    n2 = jnp.sum(y * y, axis=1, keepdims=True)
    n = jnp.sqrt(n2)
    out_ref[...] = y / jnp.where(n == 0, 1.0, n)


def _w_transform(self_rows, agg_rows, w_pad, w1_t, w2_t, w_bias):
    blk = 1024
    grid = (BPAD // blk,)
    return pl.pallas_call(
        _w_body,
        grid=grid,
        in_specs=[
            pl.BlockSpec((blk, F), lambda i: (i, 0)),
            pl.BlockSpec((blk, F), lambda i: (i, 0)),
            pl.BlockSpec((blk, T), lambda i: (i, 0)),
            pl.BlockSpec((F, F), lambda i: (0, 0)),
            pl.BlockSpec((F, F), lambda i: (0, 0)),
            pl.BlockSpec((1, F), lambda i: (0, 0)),
        ],
        out_specs=pl.BlockSpec((blk, F), lambda i: (i, 0)),
        out_shape=jax.ShapeDtypeStruct((BPAD, F), jnp.float32),
    )(self_rows, agg_rows, w_pad, w1_t, w2_t, w_bias)


@jax.jit
def kernel(h, nodeset, nb_nodes, nb_weights, Q_weight, Q_bias, W_weight,
           W_bias):
    n_batch = nodeset.shape[0]
    pad = BPAD - n_batch

    hq = _q_transform(h, Q_weight.T, Q_bias.reshape(1, F))

    nbidx_flat = jnp.pad(nb_nodes.astype(jnp.int32),
                         ((0, pad), (0, 0))).reshape(-1)
    w_flat = jnp.pad(nb_weights, ((0, pad), (0, 0))).reshape(-1)
    ns_pad = jnp.pad(nodeset.astype(jnp.int32), (0, pad))

    agg, self_rows = _sc_gather_agg(hq, h, nbidx_flat, w_flat, ns_pad)

    w_pad = jnp.pad(nb_weights, ((0, pad), (0, 0)))
    out = _w_transform(self_rows, agg, w_pad, W_weight[:, :F].T,
                       W_weight[:, F:].T, W_bias.reshape(1, F))
    return out[:n_batch]


# speedup vs baseline: 1.2308x; 1.0574x over previous
import functools

import jax
import jax.numpy as jnp
from jax import lax
from jax.experimental import pallas as pl
from jax.experimental.pallas import tpu as pltpu
from jax.experimental.pallas import tpu_sc as plsc

F = 128
NC, NS = 2, 16
NW = NC * NS
BPAD = 10240
T = 32
BPW = BPAD // NW
CH = 8
NCHUNK = BPW // CH


def _q_body(h_ref, qt_ref, qb_ref, out_ref):
    y = jnp.dot(h_ref[...], qt_ref[...], preferred_element_type=jnp.float32)
    y = y + qb_ref[...]
    out_ref[...] = jnp.where(y >= 0, y, 0.01 * y)


def _q_transform(h, q_t, q_bias):
    n = h.shape[0]
    blk = 1000
    grid = (n // blk,)
    return pl.pallas_call(
        _q_body,
        grid=grid,
        in_specs=[
            pl.BlockSpec((blk, F), lambda i: (i, 0)),
            pl.BlockSpec((F, F), lambda i: (0, 0)),
            pl.BlockSpec((1, F), lambda i: (0, 0)),
        ],
        out_specs=pl.BlockSpec((blk, F), lambda i: (i, 0)),
        out_shape=jax.ShapeDtypeStruct((n, F), jnp.float32),
    )(h, q_t, q_bias)


def _bcast_lane(vec, lane):
    return lax.gather(
        vec,
        jnp.full((16, 1), lane, jnp.int32),
        lax.GatherDimensionNumbers(
            offset_dims=(), collapsed_slice_dims=(0,),
            start_index_map=(0,)),
        slice_sizes=(1,),
        mode=lax.GatherScatterMode.PROMISE_IN_BOUNDS)


def _sc_body(hq_hbm, h_hbm, nbidx_hbm, w_hbm, ns_hbm,
             agg_hbm, self_hbm,
             idx_v, w_v, ns_v, rows0_v, rows1_v, out_v,
             self0_v, self1_v, sem0, sem1, ssem0, ssem1):
    wid = lax.axis_index("s") * NC + lax.axis_index("c")
    ebase = wid * (BPW * T)
    rbase = wid * BPW
    pltpu.sync_copy(nbidx_hbm.at[pl.ds(ebase, BPW * T)], idx_v)
    pltpu.sync_copy(w_hbm.at[pl.ds(ebase, BPW * T)], w_v)
    pltpu.sync_copy(ns_hbm.at[pl.ds(rbase, BPW)], ns_v)

    rows_bufs = (rows0_v, rows1_v)
    self_bufs = (self0_v, self1_v)
    sems = (sem0, sem1)
    ssems = (ssem0, ssem1)
    nj = (CH * T) // 128

    def fire(g, b):
        cbase = pl.multiple_of(g * (CH * T), CH * T)
        for j in range(nj):
            pltpu.async_copy(
                hq_hbm.at[idx_v.at[pl.ds(cbase + j * 128, 128)]],
                rows_bufs[b].at[pl.ds(j * 128, 128)],
                sems[b])
        pltpu.async_copy(
            h_hbm.at[ns_v.at[pl.ds(pl.multiple_of(g * CH, CH), CH)]],
            self_bufs[b], ssems[b])

    def drain(b):
        for j in range(nj):
            pltpu.make_async_copy(
                hq_hbm.at[idx_v.at[pl.ds(j * 128, 128)]],
                rows_bufs[b].at[pl.ds(j * 128, 128)],
                sems[b]).wait()
        pltpu.make_async_copy(
            h_hbm.at[ns_v.at[pl.ds(0, CH)]], self_bufs[b], ssems[b]).wait()

    def compute(g, b):
        rows = rows_bufs[b]
        cbase = g * (CH * T)

        def row_body(i, _):
            eb = cbase + i * T
            wv0 = w_v[pl.ds(eb, 16)]
            wv1 = w_v[pl.ds(eb + 16, 16)]
            acc = [jnp.zeros((16,), jnp.float32) for _ in range(8)]
            for t in range(T):
                wb = _bcast_lane(wv0 if t < 16 else wv1, t % 16)
                r = i * T + t
                for k in range(8):
                    acc[k] = acc[k] + wb * rows[r, pl.ds(k * 16, 16)]
            for k in range(8):
                out_v[i, pl.ds(k * 16, 16)] = acc[k]
            return 0

        lax.fori_loop(0, CH, row_body, 0)
        pltpu.sync_copy(out_v, agg_hbm.at[pl.ds(rbase + g * CH, CH)])
        pltpu.sync_copy(
            self_bufs[b], self_hbm.at[pl.ds(rbase + g * CH, CH)])

    fire(0, 0)

    def pair_body(p, _):
        g0 = p * 2
        fire(g0 + 1, 1)
        drain(0)
        compute(g0, 0)

        @pl.when(g0 + 2 < NCHUNK)
        def _():
            fire(g0 + 2, 0)

        drain(1)
        compute(g0 + 1, 1)
        return 0

    lax.fori_loop(0, NCHUNK // 2, pair_body, 0)


def _sc_gather_agg(hq, h, nbidx_flat, w_flat, ns_pad):
    mesh = plsc.VectorSubcoreMesh(core_axis_name="c", subcore_axis_name="s")
    kern = functools.partial(
        pl.kernel,
        mesh=mesh,
        out_type=(
            jax.ShapeDtypeStruct((BPAD, F), jnp.float32),
            jax.ShapeDtypeStruct((BPAD, F), jnp.float32),
        ),
        scratch_types=[
            pltpu.VMEM((BPW * T,), jnp.int32),
            pltpu.VMEM((BPW * T,), jnp.float32),
            pltpu.VMEM((BPW,), jnp.int32),
            pltpu.VMEM((CH * T, F), jnp.float32),
            pltpu.VMEM((CH * T, F), jnp.float32),
            pltpu.VMEM((CH, F), jnp.float32),
            pltpu.VMEM((CH, F), jnp.float32),
            pltpu.VMEM((CH, F), jnp.float32),
            pltpu.SemaphoreType.DMA,
            pltpu.SemaphoreType.DMA,
            pltpu.SemaphoreType.DMA,
            pltpu.SemaphoreType.DMA,
        ],
    )(_sc_body)
    return kern(hq, h, nbidx_flat, w_flat, ns_pad)


def _w_body(self_ref, agg_ref, w_ref, w1t_ref, w2t_ref, wb_ref, out_ref):
    wsum = jnp.sum(w_ref[...], axis=1, keepdims=True)
    wsum = jnp.where(wsum == 0, 1.0, wsum)
    y = jnp.dot(self_ref[...], w1t_ref[...], preferred_element_type=jnp.float32)
    y2 = jnp.dot(agg_ref[...], w2t_ref[...], preferred_element_type=jnp.float32)
    y = y + y2 / wsum + wb_ref[...]
    y = jnp.where(y >= 0, y, 0.01 * y)
    n2 = jnp.sum(y * y, axis=1, keepdims=True)
    n = jnp.sqrt(n2)
    out_ref[...] = y / jnp.where(n == 0, 1.0, n)


def _w_transform(self_rows, agg_rows, w_pad, w1_t, w2_t, w_bias):
    blk = 1024
    grid = (BPAD // blk,)
    return pl.pallas_call(
        _w_body,
        grid=grid,
        in_specs=[
            pl.BlockSpec((blk, F), lambda i: (i, 0)),
            pl.BlockSpec((blk, F), lambda i: (i, 0)),
            pl.BlockSpec((blk, T), lambda i: (i, 0)),
            pl.BlockSpec((F, F), lambda i: (0, 0)),
            pl.BlockSpec((F, F), lambda i: (0, 0)),
            pl.BlockSpec((1, F), lambda i: (0, 0)),
        ],
        out_specs=pl.BlockSpec((blk, F), lambda i: (i, 0)),
        out_shape=jax.ShapeDtypeStruct((BPAD, F), jnp.float32),
    )(self_rows, agg_rows, w_pad, w1_t, w2_t, w_bias)


@jax.jit
def kernel(h, nodeset, nb_nodes, nb_weights, Q_weight, Q_bias, W_weight,
           W_bias):
    n_batch = nodeset.shape[0]
    pad = BPAD - n_batch

    hq = _q_transform(h, Q_weight.T, Q_bias.reshape(1, F))

    nbidx_flat = jnp.pad(nb_nodes.astype(jnp.int32),
                         ((0, pad), (0, 0))).reshape(-1)
    w_flat = jnp.pad(nb_weights, ((0, pad), (0, 0))).reshape(-1)
    ns_pad = jnp.pad(nodeset.astype(jnp.int32), (0, pad))

    agg, self_rows = _sc_gather_agg(hq, h, nbidx_flat, w_flat, ns_pad)

    w_pad = jnp.pad(nb_weights, ((0, pad), (0, 0)))
    out = _w_transform(self_rows, agg, w_pad, W_weight[:, :F].T,
                       W_weight[:, F:].T, W_bias.reshape(1, F))
    return out[:n_batch]


# speedup vs baseline: 1.3755x; 1.1175x over previous
import functools

import jax
import jax.numpy as jnp
from jax import lax
from jax.experimental import pallas as pl
from jax.experimental.pallas import tpu as pltpu
from jax.experimental.pallas import tpu_sc as plsc

F = 128
NC, NS = 2, 16
NW = NC * NS
BPAD = 10240
T = 32
CH = 8
R0 = 544
R1 = (BPAD - NS * R0) // NS


def _q_body(h_ref, qt_ref, qb_ref, out_ref):
    y = jnp.dot(h_ref[...], qt_ref[...], preferred_element_type=jnp.float32)
    y = y + qb_ref[...]
    out_ref[...] = jnp.where(y >= 0, y, 0.01 * y)


def _q_transform(h, q_t, q_bias):
    n = h.shape[0]
    blk = 1000
    grid = (n // blk,)
    return pl.pallas_call(
        _q_body,
        grid=grid,
        in_specs=[
            pl.BlockSpec((blk, F), lambda i: (i, 0)),
            pl.BlockSpec((F, F), lambda i: (0, 0)),
            pl.BlockSpec((1, F), lambda i: (0, 0)),
        ],
        out_specs=pl.BlockSpec((blk, F), lambda i: (i, 0)),
        out_shape=jax.ShapeDtypeStruct((n, F), jnp.float32),
    )(h, q_t, q_bias)


def _bcast_lane(vec, lane):
    return lax.gather(
        vec,
        jnp.full((16, 1), lane, jnp.int32),
        lax.GatherDimensionNumbers(
            offset_dims=(), collapsed_slice_dims=(0,),
            start_index_map=(0,)),
        slice_sizes=(1,),
        mode=lax.GatherScatterMode.PROMISE_IN_BOUNDS)


def _sc_body(hq_hbm, h_hbm, nbidx_hbm, w_hbm, ns_hbm,
             agg_hbm, self_hbm,
             idx_v, w_v, ns_v, rows0_v, rows1_v, out_v,
             self0_v, self1_v, sem0, sem1, ssem0, ssem1):
    c = lax.axis_index("c")
    s = lax.axis_index("s")
    rbase = jnp.where(c == 0, s * R0, NS * R0 + s * R1)
    ebase = rbase * T
    nchunk = jnp.where(c == 0, R0 // CH, R1 // CH)

    @pl.when(c == 0)
    def _():
        pltpu.sync_copy(nbidx_hbm.at[pl.ds(s * (R0 * T), R0 * T)], idx_v)
        pltpu.sync_copy(w_hbm.at[pl.ds(s * (R0 * T), R0 * T)], w_v)
        pltpu.sync_copy(ns_hbm.at[pl.ds(s * R0, R0)], ns_v)

    @pl.when(c == 1)
    def _():
        pltpu.sync_copy(
            nbidx_hbm.at[pl.ds(NS * (R0 * T) + s * (R1 * T), R1 * T)],
            idx_v.at[pl.ds(0, R1 * T)])
        pltpu.sync_copy(w_hbm.at[pl.ds(NS * (R0 * T) + s * (R1 * T), R1 * T)],
                        w_v.at[pl.ds(0, R1 * T)])
        pltpu.sync_copy(ns_hbm.at[pl.ds(NS * R0 + s * R1, R1)],
                        ns_v.at[pl.ds(0, R1)])

    rows_bufs = (rows0_v, rows1_v)
    self_bufs = (self0_v, self1_v)
    sems = (sem0, sem1)
    ssems = (ssem0, ssem1)
    nj = (CH * T) // 128

    def fire(g, b):
        cbase = pl.multiple_of(g * (CH * T), CH * T)
        for j in range(nj):
            pltpu.async_copy(
                hq_hbm.at[idx_v.at[pl.ds(cbase + j * 128, 128)]],
                rows_bufs[b].at[pl.ds(j * 128, 128)],
                sems[b])
        pltpu.async_copy(
            h_hbm.at[ns_v.at[pl.ds(pl.multiple_of(g * CH, CH), CH)]],
            self_bufs[b], ssems[b])

    def drain(b):
        for j in range(nj):
            pltpu.make_async_copy(
                hq_hbm.at[idx_v.at[pl.ds(j * 128, 128)]],
                rows_bufs[b].at[pl.ds(j * 128, 128)],
                sems[b]).wait()
        pltpu.make_async_copy(
            h_hbm.at[ns_v.at[pl.ds(0, CH)]], self_bufs[b], ssems[b]).wait()

    def compute(g, b):
        rows = rows_bufs[b]
        cbase = g * (CH * T)

        def row_body(i, _):
            eb = cbase + i * T
            wv0 = w_v[pl.ds(eb, 16)]
            wv1 = w_v[pl.ds(eb + 16, 16)]
            acc = [jnp.zeros((16,), jnp.float32) for _ in range(8)]
            for t in range(T):
                wb = _bcast_lane(wv0 if t < 16 else wv1, t % 16)
                r = i * T + t
                for k in range(8):
                    acc[k] = acc[k] + wb * rows[r, pl.ds(k * 16, 16)]
            for k in range(8):
                out_v[i, pl.ds(k * 16, 16)] = acc[k]
            return 0

        lax.fori_loop(0, CH, row_body, 0)
        pltpu.sync_copy(out_v, agg_hbm.at[pl.ds(rbase + g * CH, CH)])
        pltpu.sync_copy(
            self_bufs[b], self_hbm.at[pl.ds(rbase + g * CH, CH)])

    fire(0, 0)

    def pair_body(p, _):
        g0 = p * 2
        fire(g0 + 1, 1)
        drain(0)
        compute(g0, 0)

        @pl.when(g0 + 2 < nchunk)
        def _():
            fire(g0 + 2, 0)

        drain(1)
        compute(g0 + 1, 1)
        return 0

    lax.fori_loop(0, nchunk // 2, pair_body, 0)


def _sc_gather_agg(hq, h, nbidx_flat, w_flat, ns_pad):
    mesh = plsc.VectorSubcoreMesh(core_axis_name="c", subcore_axis_name="s")
    kern = functools.partial(
        pl.kernel,
        mesh=mesh,
        out_type=(
            jax.ShapeDtypeStruct((BPAD, F), jnp.float32),
            jax.ShapeDtypeStruct((BPAD, F), jnp.float32),
        ),
        scratch_types=[
            pltpu.VMEM((R0 * T,), jnp.int32),
            pltpu.VMEM((R0 * T,), jnp.float32),
            pltpu.VMEM((R0,), jnp.int32),
            pltpu.VMEM((CH * T, F), jnp.float32),
            pltpu.VMEM((CH * T, F), jnp.float32),
            pltpu.VMEM((CH, F), jnp.float32),
            pltpu.VMEM((CH, F), jnp.float32),
            pltpu.VMEM((CH, F), jnp.float32),
            pltpu.SemaphoreType.DMA,
            pltpu.SemaphoreType.DMA,
            pltpu.SemaphoreType.DMA,
            pltpu.SemaphoreType.DMA,
        ],
    )(_sc_body)
    return kern(hq, h, nbidx_flat, w_flat, ns_pad)


def _w_body(self_ref, agg_ref, w_ref, w1t_ref, w2t_ref, wb_ref, out_ref):
    wsum = jnp.sum(w_ref[...], axis=1, keepdims=True)
    wsum = jnp.where(wsum == 0, 1.0, wsum)
    y = jnp.dot(self_ref[...], w1t_ref[...], preferred_element_type=jnp.float32)
    y2 = jnp.dot(agg_ref[...], w2t_ref[...], preferred_element_type=jnp.float32)
    y = y + y2 / wsum + wb_ref[...]
    y = jnp.where(y >= 0, y, 0.01 * y)
    n2 = jnp.sum(y * y, axis=1, keepdims=True)
    n = jnp.sqrt(n2)
    out_ref[...] = y / jnp.where(n == 0, 1.0, n)


def _w_transform(self_rows, agg_rows, w_pad, w1_t, w2_t, w_bias):
    blk = 1024
    grid = (BPAD // blk,)
    return pl.pallas_call(
        _w_body,
        grid=grid,
        in_specs=[
            pl.BlockSpec((blk, F), lambda i: (i, 0)),
            pl.BlockSpec((blk, F), lambda i: (i, 0)),
            pl.BlockSpec((blk, T), lambda i: (i, 0)),
            pl.BlockSpec((F, F), lambda i: (0, 0)),
            pl.BlockSpec((F, F), lambda i: (0, 0)),
            pl.BlockSpec((1, F), lambda i: (0, 0)),
        ],
        out_specs=pl.BlockSpec((blk, F), lambda i: (i, 0)),
        out_shape=jax.ShapeDtypeStruct((BPAD, F), jnp.float32),
    )(self_rows, agg_rows, w_pad, w1_t, w2_t, w_bias)


@jax.jit
def kernel(h, nodeset, nb_nodes, nb_weights, Q_weight, Q_bias, W_weight,
           W_bias):
    n_batch = nodeset.shape[0]
    pad = BPAD - n_batch

    hq = _q_transform(h, Q_weight.T, Q_bias.reshape(1, F))

    nbidx_flat = jnp.pad(nb_nodes.astype(jnp.int32),
                         ((0, pad), (0, 0))).reshape(-1)
    w_flat = jnp.pad(nb_weights, ((0, pad), (0, 0))).reshape(-1)
    ns_pad = jnp.pad(nodeset.astype(jnp.int32), (0, pad))

    agg, self_rows = _sc_gather_agg(hq, h, nbidx_flat, w_flat, ns_pad)

    w_pad = jnp.pad(nb_weights, ((0, pad), (0, 0)))
    out = _w_transform(self_rows, agg, w_pad, W_weight[:, :F].T,
                       W_weight[:, F:].T, W_bias.reshape(1, F))
    return out[:n_batch]
